# baseline (speedup 1.0000x reference)
import functools

import jax
import jax.numpy as jnp
from jax import lax
from jax.experimental import pallas as pl
from jax.experimental.pallas import tpu as pltpu
from jax.experimental.pallas import tpu_sc as plsc

_VOCAB = 1000000
_EMB = 64
_B = 4096
_L = 200
_SCALE = 8.0

_NC = 2
_NS = 16
_NW = _NC * _NS
_BPW = _B // _NW
_H0 = 104
_H1 = _L - _H0
_NH = 2 * _BPW
_DEPTH = 4


def _emb_body(tokens_hbm, table_hbm, out_hbm, idx_v, gbuf, obuf, *sems):
    gsems = sems[:_DEPTH]
    osems = sems[_DEPTH:]

    wid = lax.axis_index("s") * _NC + lax.axis_index("c")
    b0 = wid * _BPW

    pltpu.sync_copy(tokens_hbm.at[pl.ds(b0, _BPW)], idx_v)

    def _geom(h_static_parity):
        return (0, _H0) if h_static_parity == 0 else (_H0, _H1)

    def start_gather(h, k):
        off, sz = _geom(k % 2)
        row = h // 2
        pltpu.async_copy(
            table_hbm.at[idx_v.at[row, pl.ds(off, sz)]],
            gbuf.at[k, pl.ds(0, sz)],
            gsems[k],
        )

    def wait_gather(h, k):
        off, sz = _geom(k % 2)
        row = h // 2
        pltpu.make_async_copy(
            table_hbm.at[idx_v.at[row, pl.ds(off, sz)]],
            gbuf.at[k, pl.ds(0, sz)],
            gsems[k],
        ).wait()

    def start_out(h, k):
        off, sz = _geom(k % 2)
        row = h // 2
        pltpu.async_copy(
            obuf.at[k, pl.ds(0, sz)],
            out_hbm.at[b0 + row, pl.ds(off, sz)],
            osems[k],
        )

    def wait_out(h, k):
        off, sz = _geom(k % 2)
        row = h // 2
        pltpu.make_async_copy(
            obuf.at[k, pl.ds(0, sz)],
            out_hbm.at[b0 + row, pl.ds(off, sz)],
            osems[k],
        ).wait()

    for k in range(_DEPTH):
        start_gather(k, k)

    def round_body(i, carry):
        for k in range(_DEPTH):
            h = _DEPTH * i + k
            wait_gather(h, k)

            @pl.when(h >= _DEPTH)
            def _():
                wait_out(h - _DEPTH, k)

            sz = _geom(k % 2)[1]

            def scale_row(r, c):
                for j in range(_EMB // 16):
                    sl = pl.ds(j * 16, 16)
                    obuf[k, r, sl] = gbuf[k, r, sl] * _SCALE
                return c

            lax.fori_loop(0, sz, scale_row, 0, unroll=8)

            start_out(h, k)

            @pl.when(h + _DEPTH < _NH)
            def _():
                start_gather(h + _DEPTH, k)

        return carry

    lax.fori_loop(0, _NH // _DEPTH, round_body, 0)

    for k in range(_DEPTH):
        wait_out(_NH - _DEPTH + k, k)


@jax.jit
def _embed(tokens, table):
    run = functools.partial(
        pl.kernel,
        mesh=plsc.VectorSubcoreMesh(core_axis_name="c", subcore_axis_name="s"),
        out_type=jax.ShapeDtypeStruct((_B, _L, _EMB), jnp.float32),
        scratch_types=[
            pltpu.VMEM((_BPW, _L), jnp.int32),
            pltpu.VMEM((_DEPTH, _H0, _EMB), jnp.float32),
            pltpu.VMEM((_DEPTH, _H0, _EMB), jnp.float32),
        ]
        + [pltpu.SemaphoreType.DMA] * (2 * _DEPTH),
        compiler_params=pltpu.CompilerParams(use_tc_tiling_on_sc=False),
    )(_emb_body)
    return run(tokens, table)


def kernel(tokens, table):
    return _embed(tokens, table)
